# Initial kernel scaffold; baseline (speedup 1.0000x reference)
import jax
import jax.numpy as jnp
from jax.experimental import pallas as pl


def kernel(feat_static_cat, feat_static_real, feat_dynamic_cat, feat_dynamic_real, static_table0, static_table1, dyn_table0):
    raise NotImplementedError("write your pallas kernel here")


# speedup vs baseline: 2.2533x; 2.2533x over previous
import functools

import jax
import jax.numpy as jnp
from jax import lax
from jax.experimental import pallas as pl
from jax.experimental.pallas import tpu as pltpu
from jax.experimental.pallas import tpu_sc as plsc

B = 4096
T = 200
D_OUT = 112
BT = B * T
NW = 32
CH = 128
G = 8
N_CH = BT // CH
CH_PER_W = N_CH // NW
NG = CH_PER_W // G
SB = B // NW
BB = 8


def _sc_gather(idx2d, sidx0, sidx1, dyn_table, st0, st1):
  mesh = plsc.VectorSubcoreMesh(core_axis_name="c", subcore_axis_name="s")

  @functools.partial(
      pl.kernel,
      out_type=(
          jax.ShapeDtypeStruct((BT, 32), jnp.float32),
          jax.ShapeDtypeStruct((B, 32), jnp.float32),
          jax.ShapeDtypeStruct((B, 32), jnp.float32),
      ),
      mesh=mesh,
      compiler_params=pltpu.CompilerParams(use_tc_tiling_on_sc=False),
      scratch_types=[
          pltpu.VMEM((G, CH), jnp.int32),
          pltpu.VMEM((G * CH, 32), jnp.float32),
          pltpu.VMEM((SB,), jnp.int32),
          pltpu.VMEM((SB, 32), jnp.float32),
          pltpu.SemaphoreType.DMA,
      ],
  )
  def k(idx_hbm, s0_hbm, s1_hbm, tbl_hbm, t0_hbm, t1_hbm,
        dyn_out, es0_out, es1_out, idx_v, rows_v, sidx_v, srows_v, sem):
    wid = lax.axis_index("s") * 2 + lax.axis_index("c")
    c0 = wid * CH_PER_W

    def group(g, carry):
      pltpu.sync_copy(idx_hbm.at[pl.ds(c0 + g * G, G)], idx_v)
      cps = [
          pltpu.async_copy(tbl_hbm.at[idx_v.at[j]],
                           rows_v.at[pl.ds(j * CH, CH)], sem)
          for j in range(G)
      ]
      for cp in cps:
        cp.wait()
      pltpu.sync_copy(rows_v, dyn_out.at[pl.ds((c0 + g * G) * CH, G * CH)])
      return carry

    lax.fori_loop(0, NG, group, 0)

    b0 = wid * SB
    pltpu.sync_copy(s0_hbm.at[pl.ds(b0, SB)], sidx_v)
    pltpu.async_copy(t0_hbm.at[sidx_v], srows_v, sem).wait()
    pltpu.sync_copy(srows_v, es0_out.at[pl.ds(b0, SB)])
    pltpu.sync_copy(s1_hbm.at[pl.ds(b0, SB)], sidx_v)
    pltpu.async_copy(t1_hbm.at[sidx_v], srows_v, sem).wait()
    pltpu.sync_copy(srows_v, es1_out.at[pl.ds(b0, SB)])

  return k(idx2d, sidx0, sidx1, dyn_table, st0, st1)


def _tc_assemble(es0, es1, sreal, emb_dyn, dyn_real2d):
  def body(s0_ref, s1_ref, sr_ref, ed_ref, dr_ref, out_ref):
    stat = jnp.concatenate([s0_ref[...], s1_ref[...], sr_ref[...]], axis=-1)
    statb = jnp.broadcast_to(stat[:, None, :], (BB, T, 72))
    statb = statb.reshape(BB * T, 72)
    out_ref[...] = jnp.concatenate([statb, ed_ref[...], dr_ref[...]],
                                   axis=-1)

  return pl.pallas_call(
      body,
      grid=(B // BB,),
      out_shape=jax.ShapeDtypeStruct((BT, D_OUT), jnp.float32),
      in_specs=[
          pl.BlockSpec((BB, 32), lambda i: (i, 0)),
          pl.BlockSpec((BB, 32), lambda i: (i, 0)),
          pl.BlockSpec((BB, 8), lambda i: (i, 0)),
          pl.BlockSpec((BB * T, 32), lambda i: (i, 0)),
          pl.BlockSpec((BB * T, 8), lambda i: (i, 0)),
      ],
      out_specs=pl.BlockSpec((BB * T, D_OUT), lambda i: (i, 0)),
      compiler_params=pltpu.CompilerParams(
          dimension_semantics=("arbitrary",)),
  )(es0, es1, sreal, emb_dyn, dyn_real2d)


def kernel(feat_static_cat, feat_static_real, feat_dynamic_cat,
           feat_dynamic_real, static_table0, static_table1, dyn_table0):
  idx2d = feat_dynamic_cat.astype(jnp.int32).reshape(N_CH, CH)
  s0 = feat_static_cat[:, 0].astype(jnp.int32)
  s1 = feat_static_cat[:, 1].astype(jnp.int32)
  emb_dyn, es0, es1 = _sc_gather(idx2d, s0, s1, dyn_table0,
                                 static_table0, static_table1)
  dr2d = feat_dynamic_real.reshape(BT, 8)
  out = _tc_assemble(es0, es1, feat_static_real, emb_dyn, dr2d)
  return out.reshape(B, T, D_OUT)
